# baseline (speedup 1.0000x reference)
import functools

import jax
import jax.numpy as jnp
from jax import lax
from jax.experimental import pallas as pl
from jax.experimental.pallas import tpu as pltpu
from jax.experimental.pallas import tpu_sc as plsc

B = 8192
P = 32
DIM = 128
NC = 2
NS = 16
NW = NC * NS
BPW = B // NW
NBUF = 8


def _softplus(t):
  w = jnp.exp(-jnp.abs(t))
  s = w / (2.0 + w)
  s2 = s * s
  ln1p = 2.0 * s * (1.0 + s2 * (1.0 / 3.0 + s2 * (0.2 + s2 * (1.0 / 7.0 + s2 * (1.0 / 9.0)))))
  return jnp.maximum(t, 0.0) + ln1p


def _sc_body(inputs_hbm, path_hbm, len_hbm, code_hbm, wembed_hbm, whs_hbm,
             out_hbm, inp_idx, path_v, code_v, len_v, hid_v, rowbuf,
             acc_tot, acc_cnt, outstage, sem_h0, sem_h1, sems):
  wid = lax.axis_index("s") * NC + lax.axis_index("c")
  base = wid * BPW

  pltpu.sync_copy(inputs_hbm.at[pl.ds(base, BPW)], inp_idx)
  pltpu.sync_copy(path_hbm.at[pl.ds(base * P, BPW * P)], path_v)
  pltpu.sync_copy(code_hbm.at[pl.ds(base * P, BPW * P)], code_v)
  pltpu.sync_copy(len_hbm.at[pl.ds(base, BPW)], len_v)

  h0 = pltpu.async_copy(wembed_hbm.at[inp_idx.at[pl.ds(0, 128)]],
                        hid_v.at[pl.ds(0, 128)], sem_h0)
  h1 = pltpu.async_copy(wembed_hbm.at[inp_idx.at[pl.ds(128, 128)]],
                        hid_v.at[pl.ds(128, 128)], sem_h1)
  h0.wait()
  h1.wait()

  acc_tot[...] = jnp.zeros((16,), jnp.float32)
  acc_cnt[...] = jnp.zeros((16,), jnp.float32)

  lanes = lax.iota(jnp.int32, 16)

  def _len_of(i):
    return plsc.load_gather(len_v, [jnp.full((16,), i, jnp.int32)])[0]

  def issue_gathers(i, b, len_i):
    @pl.when(len_i > 0)
    def _():
      pltpu.async_copy(whs_hbm.at[path_v.at[pl.ds(i * P, 16)]],
                       rowbuf.at[b, pl.ds(0, 16)], sems[2 * b])

    @pl.when(len_i > 16)
    def _():
      pltpu.async_copy(whs_hbm.at[path_v.at[pl.ds(i * P + 16, 16)]],
                       rowbuf.at[b, pl.ds(16, 16)], sems[2 * b + 1])

  def wait_gathers(i, b, len_i):
    @pl.when(len_i > 0)
    def _():
      pltpu.make_async_copy(whs_hbm.at[path_v.at[pl.ds(i * P, 16)]],
                            rowbuf.at[b, pl.ds(0, 16)], sems[2 * b]).wait()

    @pl.when(len_i > 16)
    def _():
      pltpu.make_async_copy(whs_hbm.at[path_v.at[pl.ds(i * P + 16, 16)]],
                            rowbuf.at[b, pl.ds(16, 16)], sems[2 * b + 1]).wait()

  for b in range(NBUF):
    issue_gathers(b, b, _len_of(b))

  def compute_example(e, buf):
    e_splat = jnp.full((16,), e, jnp.int32)
    len_vec = plsc.load_gather(len_v, [e_splat])
    len_e = len_vec[0]
    ngrp = (len_e + 15) // 16

    def gbody(g, carry):
      rows = lanes + g * 16
      zero = jnp.zeros((16,), jnp.float32)
      hs = [plsc.load_gather(hid_v, [e_splat, lanes + s * 16])
            for s in range(8)]
      logit = zero
      for k in range(16):
        r_splat = jnp.full((16,), g * 16 + k, jnp.int32)
        a0 = zero
        a1 = zero
        for s in range(8):
          vals = plsc.load_gather(buf, [r_splat, lanes + s * 16])
          if s % 2 == 0:
            a0 = a0 + vals * hs[s]
          else:
            a1 = a1 + vals * hs[s]
        tot = jnp.sum(a0 + a1)
        logit = jnp.where(lanes == k, tot, logit)
      codef = plsc.load_gather(
          code_v, [jnp.full((16,), e * P, jnp.int32) + rows]).astype(jnp.float32)
      t = logit * (1.0 - 2.0 * codef)
      sp = _softplus(t)
      valid = rows < len_vec
      acc_tot[...] = acc_tot[...] + jnp.where(valid, sp, 0.0)
      acc_cnt[...] = acc_cnt[...] + jnp.where(valid, 1.0, 0.0)
      return carry

    lax.fori_loop(0, ngrp, gbody, 0)

  def chunk_body(i, carry):
    for b in range(NBUF):
      e = i * NBUF + b
      len_e = _len_of(e)
      wait_gathers(e, b, len_e)
      compute_example(e, rowbuf.at[b])
      en = e + NBUF

      @pl.when(en < BPW)
      def _():
        issue_gathers(en, b, _len_of(en))
    return carry

  lax.fori_loop(0, BPW // NBUF, chunk_body, 0)

  outstage[0, :] = acc_tot[...]
  outstage[1, :] = acc_cnt[...]
  pltpu.sync_copy(outstage, out_hbm.at[wid])


@jax.jit
def _hs_loss(inputs, target_path, target_path_len, target_code, W_embed, W_hs):
  mesh = plsc.VectorSubcoreMesh(core_axis_name="c", subcore_axis_name="s")
  parts = pl.kernel(
      _sc_body,
      out_type=jax.ShapeDtypeStruct((NW, 2, 16), jnp.float32),
      mesh=mesh,
      compiler_params=pltpu.CompilerParams(needs_layout_passes=False),
      scratch_types=[
          pltpu.VMEM((BPW,), jnp.int32),
          pltpu.VMEM((BPW * P,), jnp.int32),
          pltpu.VMEM((BPW * P,), jnp.int32),
          pltpu.VMEM((BPW,), jnp.int32),
          pltpu.VMEM((BPW, DIM), jnp.float32),
          pltpu.VMEM((NBUF, P, DIM), jnp.float32),
          pltpu.VMEM((16,), jnp.float32),
          pltpu.VMEM((16,), jnp.float32),
          pltpu.VMEM((2, 16), jnp.float32),
          pltpu.SemaphoreType.DMA,
          pltpu.SemaphoreType.DMA,
          [pltpu.SemaphoreType.DMA] * (2 * NBUF),
      ],
  )(inputs, target_path.reshape(-1), target_path_len, target_code.reshape(-1),
    W_embed, W_hs)
  total = jnp.sum(parts[:, 0, :])
  count = jnp.sum(parts[:, 1, :])
  return total / count


def kernel(inputs, target, target_path, target_path_len, target_code,
           W_embed, W_hs):
  del target
  return _hs_loss(inputs.astype(jnp.int32), target_path.astype(jnp.int32),
                  target_path_len.astype(jnp.int32),
                  target_code.astype(jnp.int32), W_embed, W_hs)


# speedup vs baseline: 1.0664x; 1.0664x over previous
import functools

import jax
import jax.numpy as jnp
from jax import lax
from jax.experimental import pallas as pl
from jax.experimental.pallas import tpu as pltpu
from jax.experimental.pallas import tpu_sc as plsc

B = 8192
P = 32
DIM = 128
NC = 2
NS = 16
NW = NC * NS
BPW = B // NW
NBUF = 8


def _softplus(t):
  w = jnp.exp(-jnp.abs(t))
  s = w / (2.0 + w)
  s2 = s * s
  ln1p = 2.0 * s * (1.0 + s2 * (1.0 / 3.0 + s2 * (0.2 + s2 * (1.0 / 7.0 + s2 * (1.0 / 9.0)))))
  return jnp.maximum(t, 0.0) + ln1p


def _sc_body(inputs_hbm, path_hbm, len_hbm, code_hbm, wembed_hbm, whs_hbm,
             out_hbm, inp_idx, path_v, code_v, len_v, hid_v, rowbuf,
             acc_tot, acc_cnt, outstage, sem_h0, sem_h1, sems):
  wid = lax.axis_index("s") * NC + lax.axis_index("c")
  base = wid * BPW

  pltpu.sync_copy(inputs_hbm.at[pl.ds(base, BPW)], inp_idx)
  pltpu.sync_copy(path_hbm.at[pl.ds(base * P, BPW * P)], path_v)
  pltpu.sync_copy(code_hbm.at[pl.ds(base * P, BPW * P)], code_v)
  pltpu.sync_copy(len_hbm.at[pl.ds(base, BPW)], len_v)

  h0 = pltpu.async_copy(wembed_hbm.at[inp_idx.at[pl.ds(0, 128)]],
                        hid_v.at[pl.ds(0, 128)], sem_h0)
  h1 = pltpu.async_copy(wembed_hbm.at[inp_idx.at[pl.ds(128, 128)]],
                        hid_v.at[pl.ds(128, 128)], sem_h1)
  h0.wait()
  h1.wait()

  acc_tot[...] = jnp.zeros((16,), jnp.float32)
  acc_cnt[...] = jnp.zeros((16,), jnp.float32)

  lanes = lax.iota(jnp.int32, 16)

  def _len_of(i):
    return plsc.load_gather(len_v, [jnp.full((16,), i, jnp.int32)])[0]

  def issue_gathers(i, b, len_i):
    del len_i
    pltpu.async_copy(whs_hbm.at[path_v.at[pl.ds(i * P, P)]],
                     rowbuf.at[b], sems[b])

  def wait_gathers(i, b, len_i):
    del len_i
    pltpu.make_async_copy(whs_hbm.at[path_v.at[pl.ds(i * P, P)]],
                          rowbuf.at[b], sems[b]).wait()

  for b in range(NBUF):
    issue_gathers(b, b, None)

  def compute_example(e, buf):
    e_splat = jnp.full((16,), e, jnp.int32)
    len_vec = plsc.load_gather(len_v, [e_splat])
    len_e = len_vec[0]
    ngrp = (len_e + 15) // 16

    def gbody(g, carry):
      rows = lanes + g * 16
      zero = jnp.zeros((16,), jnp.float32)
      hs = [plsc.load_gather(hid_v, [e_splat, lanes + s * 16])
            for s in range(8)]
      logit = zero
      for k in range(16):
        r_splat = jnp.full((16,), g * 16 + k, jnp.int32)
        a0 = zero
        a1 = zero
        for s in range(8):
          vals = plsc.load_gather(buf, [r_splat, lanes + s * 16])
          if s % 2 == 0:
            a0 = a0 + vals * hs[s]
          else:
            a1 = a1 + vals * hs[s]
        tot = jnp.sum(a0 + a1)
        logit = jnp.where(lanes == k, tot, logit)
      codef = plsc.load_gather(
          code_v, [jnp.full((16,), e * P, jnp.int32) + rows]).astype(jnp.float32)
      t = logit * (1.0 - 2.0 * codef)
      sp = _softplus(t)
      valid = rows < len_vec
      acc_tot[...] = acc_tot[...] + jnp.where(valid, sp, 0.0)
      acc_cnt[...] = acc_cnt[...] + jnp.where(valid, 1.0, 0.0)
      return carry

    lax.fori_loop(0, ngrp, gbody, 0)

  def chunk_body(i, carry):
    for b in range(NBUF):
      e = i * NBUF + b
      wait_gathers(e, b, None)
      compute_example(e, rowbuf.at[b])
      en = e + NBUF

      @pl.when(en < BPW)
      def _():
        issue_gathers(en, b, None)
    return carry

  lax.fori_loop(0, BPW // NBUF, chunk_body, 0)

  outstage[0, :] = acc_tot[...]
  outstage[1, :] = acc_cnt[...]
  pltpu.sync_copy(outstage, out_hbm.at[wid])


@jax.jit
def _hs_loss(inputs, target_path, target_path_len, target_code, W_embed, W_hs):
  mesh = plsc.VectorSubcoreMesh(core_axis_name="c", subcore_axis_name="s")
  parts = pl.kernel(
      _sc_body,
      out_type=jax.ShapeDtypeStruct((NW, 2, 16), jnp.float32),
      mesh=mesh,
      compiler_params=pltpu.CompilerParams(needs_layout_passes=False),
      scratch_types=[
          pltpu.VMEM((BPW,), jnp.int32),
          pltpu.VMEM((BPW * P,), jnp.int32),
          pltpu.VMEM((BPW * P,), jnp.int32),
          pltpu.VMEM((BPW,), jnp.int32),
          pltpu.VMEM((BPW, DIM), jnp.float32),
          pltpu.VMEM((NBUF, P, DIM), jnp.float32),
          pltpu.VMEM((16,), jnp.float32),
          pltpu.VMEM((16,), jnp.float32),
          pltpu.VMEM((2, 16), jnp.float32),
          pltpu.SemaphoreType.DMA,
          pltpu.SemaphoreType.DMA,
          [pltpu.SemaphoreType.DMA] * (2 * NBUF),
      ],
  )(inputs, target_path.reshape(-1), target_path_len, target_code.reshape(-1),
    W_embed, W_hs)
  total = jnp.sum(parts[:, 0, :])
  count = jnp.sum(parts[:, 1, :])
  return total / count


def kernel(inputs, target, target_path, target_path_len, target_code,
           W_embed, W_hs):
  del target
  return _hs_loss(inputs.astype(jnp.int32), target_path.astype(jnp.int32),
                  target_path_len.astype(jnp.int32),
                  target_code.astype(jnp.int32), W_embed, W_hs)


# speedup vs baseline: 1.4783x; 1.3862x over previous
import functools

import jax
import jax.numpy as jnp
from jax import lax
from jax.experimental import pallas as pl
from jax.experimental.pallas import tpu as pltpu
from jax.experimental.pallas import tpu_sc as plsc

B = 8192
P = 32
DIM = 128
NC = 2
NS = 16
NW = NC * NS
BPW = B // NW
NBUF = 4


def _softplus(t):
  w = jnp.exp(-jnp.abs(t))
  s = w / (2.0 + w)
  s2 = s * s
  ln1p = 2.0 * s * (1.0 + s2 * (1.0 / 3.0 + s2 * (0.2 + s2 * (1.0 / 7.0 + s2 * (1.0 / 9.0)))))
  return jnp.maximum(t, 0.0) + ln1p


def _sc_body(inputs_hbm, path_hbm, len_hbm, code_hbm, wembed_hbm, whs_hbm,
             out_hbm, inp_idx, path_v, code_v, len_v, hid_v, rowbuf,
             acc_tot, acc_cnt, outstage, sem_h0, sem_h1, sems):
  wid = lax.axis_index("s") * NC + lax.axis_index("c")
  base = wid * BPW

  pltpu.sync_copy(inputs_hbm.at[pl.ds(base, BPW)], inp_idx)
  pltpu.sync_copy(path_hbm.at[pl.ds(base * P, BPW * P)], path_v)
  pltpu.sync_copy(code_hbm.at[pl.ds(base * P, BPW * P)], code_v)
  pltpu.sync_copy(len_hbm.at[pl.ds(base, BPW)], len_v)

  h0 = pltpu.async_copy(wembed_hbm.at[inp_idx.at[pl.ds(0, 128)]],
                        hid_v.at[pl.ds(0, 128)], sem_h0)
  h1 = pltpu.async_copy(wembed_hbm.at[inp_idx.at[pl.ds(128, 128)]],
                        hid_v.at[pl.ds(128, 128)], sem_h1)
  h0.wait()
  h1.wait()

  acc_tot[...] = jnp.zeros((16,), jnp.float32)
  acc_cnt[...] = jnp.zeros((16,), jnp.float32)

  lanes = lax.iota(jnp.int32, 16)

  def _len_of(i):
    return plsc.load_gather(len_v, [jnp.full((16,), i, jnp.int32)])[0]

  def issue_gathers(i, b, len_i):
    del len_i
    pltpu.async_copy(whs_hbm.at[path_v.at[pl.ds(i * P, P)]],
                     rowbuf.at[b], sems[b])

  def wait_gathers(i, b, len_i):
    del len_i
    pltpu.make_async_copy(whs_hbm.at[path_v.at[pl.ds(i * P, P)]],
                          rowbuf.at[b], sems[b]).wait()

  for b in range(NBUF):
    issue_gathers(b, b, None)

  def compute_example(e, buf):
    e_splat = jnp.full((16,), e, jnp.int32)
    len_vec = plsc.load_gather(len_v, [e_splat])
    len_e = len_vec[0]
    ngrp = (len_e + 15) // 16

    def gbody(g, carry):
      rows = lanes + g * 16
      zero = jnp.zeros((16,), jnp.float32)
      hs = [plsc.load_gather(hid_v, [e_splat, lanes + s * 16])
            for s in range(8)]
      logit = zero
      for k in range(16):
        r_splat = jnp.full((16,), g * 16 + k, jnp.int32)
        a0 = zero
        a1 = zero
        for s in range(8):
          vals = plsc.load_gather(buf, [r_splat, lanes + s * 16])
          if s % 2 == 0:
            a0 = a0 + vals * hs[s]
          else:
            a1 = a1 + vals * hs[s]
        tot = jnp.sum(a0 + a1)
        logit = jnp.where(lanes == k, tot, logit)
      codef = plsc.load_gather(
          code_v, [jnp.full((16,), e * P, jnp.int32) + rows]).astype(jnp.float32)
      t = logit * (1.0 - 2.0 * codef)
      sp = _softplus(t)
      valid = rows < len_vec
      acc_tot[...] = acc_tot[...] + jnp.where(valid, sp, 0.0)
      acc_cnt[...] = acc_cnt[...] + jnp.where(valid, 1.0, 0.0)
      return carry

    lax.fori_loop(0, ngrp, gbody, 0)

  def chunk_body(i, carry):
    for b in range(NBUF):
      e = i * NBUF + b
      wait_gathers(e, b, None)
      compute_example(e, rowbuf.at[b])
      en = e + NBUF

      @pl.when(en < BPW)
      def _():
        issue_gathers(en, b, None)
    return carry

  lax.fori_loop(0, BPW // NBUF, chunk_body, 0)

  outstage[0, :] = acc_tot[...]
  outstage[1, :] = acc_cnt[...]
  pltpu.sync_copy(outstage, out_hbm.at[wid])


@jax.jit
def _hs_loss(inputs, target_path, target_path_len, target_code, W_embed, W_hs):
  mesh = plsc.VectorSubcoreMesh(core_axis_name="c", subcore_axis_name="s")
  parts = pl.kernel(
      _sc_body,
      out_type=jax.ShapeDtypeStruct((NW, 2, 16), jnp.float32),
      mesh=mesh,
      compiler_params=pltpu.CompilerParams(needs_layout_passes=False),
      scratch_types=[
          pltpu.VMEM((BPW,), jnp.int32),
          pltpu.VMEM((BPW * P,), jnp.int32),
          pltpu.VMEM((BPW * P,), jnp.int32),
          pltpu.VMEM((BPW,), jnp.int32),
          pltpu.VMEM((BPW, DIM), jnp.float32),
          pltpu.VMEM((NBUF, P, DIM), jnp.float32),
          pltpu.VMEM((16,), jnp.float32),
          pltpu.VMEM((16,), jnp.float32),
          pltpu.VMEM((2, 16), jnp.float32),
          pltpu.SemaphoreType.DMA,
          pltpu.SemaphoreType.DMA,
          [pltpu.SemaphoreType.DMA] * (2 * NBUF),
      ],
  )(inputs, target_path.reshape(-1), target_path_len, target_code.reshape(-1),
    W_embed, W_hs)
  total = jnp.sum(parts[:, 0, :])
  count = jnp.sum(parts[:, 1, :])
  return total / count


def kernel(inputs, target, target_path, target_path_len, target_code,
           W_embed, W_hs):
  del target
  return _hs_loss(inputs.astype(jnp.int32), target_path.astype(jnp.int32),
                  target_path_len.astype(jnp.int32),
                  target_code.astype(jnp.int32), W_embed, W_hs)


# speedup vs baseline: 1.5148x; 1.0247x over previous
import functools

import jax
import jax.numpy as jnp
from jax import lax
from jax.experimental import pallas as pl
from jax.experimental.pallas import tpu as pltpu
from jax.experimental.pallas import tpu_sc as plsc

B = 8192
P = 32
DIM = 128
NC = 2
NS = 16
NW = NC * NS
BPW = B // NW
NBUF = 4


def _softplus(t):
  w = jnp.exp(-jnp.abs(t))
  s = w / (2.0 + w)
  s2 = s * s
  ln1p = 2.0 * s * (1.0 + s2 * (1.0 / 3.0 + s2 * (0.2 + s2 * (1.0 / 7.0 + s2 * (1.0 / 9.0)))))
  return jnp.maximum(t, 0.0) + ln1p


def _sc_body(inputs_hbm, path_hbm, len_hbm, code_hbm, wembed_hbm, whs_hbm,
             out_hbm, inp_idx, path_v, code_v, len_v, hid_v, rowbuf,
             acc_tot, acc_cnt, outstage, sem_h0, sem_h1, sems):
  wid = lax.axis_index("s") * NC + lax.axis_index("c")
  base = wid * BPW

  pltpu.sync_copy(inputs_hbm.at[pl.ds(base, BPW)], inp_idx)
  pltpu.sync_copy(path_hbm.at[pl.ds(base * P, BPW * P)], path_v)
  pltpu.sync_copy(code_hbm.at[pl.ds(base * P, BPW * P)], code_v)
  pltpu.sync_copy(len_hbm.at[pl.ds(base, BPW)], len_v)

  h0 = pltpu.async_copy(wembed_hbm.at[inp_idx.at[pl.ds(0, 128)]],
                        hid_v.at[pl.ds(0, 128)], sem_h0)
  h1 = pltpu.async_copy(wembed_hbm.at[inp_idx.at[pl.ds(128, 128)]],
                        hid_v.at[pl.ds(128, 128)], sem_h1)
  h0.wait()
  h1.wait()

  acc_tot[...] = jnp.zeros((16,), jnp.float32)
  acc_cnt[...] = jnp.zeros((16,), jnp.float32)

  lanes = lax.iota(jnp.int32, 16)

  def _len_of(i):
    return plsc.load_gather(len_v, [jnp.full((16,), i, jnp.int32)])[0]

  def issue_gathers(i, b, len_i):
    @pl.when(len_i > 16)
    def _():
      pltpu.async_copy(whs_hbm.at[path_v.at[pl.ds(i * P, P)]],
                       rowbuf.at[b], sems[b])

    @pl.when((len_i > 0) & (len_i <= 16))
    def _():
      pltpu.async_copy(whs_hbm.at[path_v.at[pl.ds(i * P, 16)]],
                       rowbuf.at[b, pl.ds(0, 16)], sems[b])

  def wait_gathers(i, b, len_i):
    @pl.when(len_i > 16)
    def _():
      pltpu.make_async_copy(whs_hbm.at[path_v.at[pl.ds(i * P, P)]],
                            rowbuf.at[b], sems[b]).wait()

    @pl.when((len_i > 0) & (len_i <= 16))
    def _():
      pltpu.make_async_copy(whs_hbm.at[path_v.at[pl.ds(i * P, 16)]],
                            rowbuf.at[b, pl.ds(0, 16)], sems[b]).wait()

  for b in range(NBUF):
    issue_gathers(b, b, _len_of(b))

  def compute_example(e, buf):
    e_splat = jnp.full((16,), e, jnp.int32)
    len_vec = plsc.load_gather(len_v, [e_splat])
    len_e = len_vec[0]
    ngrp = (len_e + 15) // 16

    def gbody(g, carry):
      rows = lanes + g * 16
      zero = jnp.zeros((16,), jnp.float32)
      hs = [plsc.load_gather(hid_v, [e_splat, lanes + s * 16])
            for s in range(8)]
      logit = zero
      for k in range(16):
        r_splat = jnp.full((16,), g * 16 + k, jnp.int32)
        a0 = zero
        a1 = zero
        for s in range(8):
          vals = plsc.load_gather(buf, [r_splat, lanes + s * 16])
          if s % 2 == 0:
            a0 = a0 + vals * hs[s]
          else:
            a1 = a1 + vals * hs[s]
        tot = jnp.sum(a0 + a1)
        logit = jnp.where(lanes == k, tot, logit)
      codef = plsc.load_gather(
          code_v, [jnp.full((16,), e * P, jnp.int32) + rows]).astype(jnp.float32)
      t = logit * (1.0 - 2.0 * codef)
      sp = _softplus(t)
      valid = rows < len_vec
      acc_tot[...] = acc_tot[...] + jnp.where(valid, sp, 0.0)
      acc_cnt[...] = acc_cnt[...] + jnp.where(valid, 1.0, 0.0)
      return carry

    lax.fori_loop(0, ngrp, gbody, 0)

  def chunk_body(i, carry):
    for b in range(NBUF):
      e = i * NBUF + b
      wait_gathers(e, b, _len_of(e))
      compute_example(e, rowbuf.at[b])
      en = e + NBUF

      @pl.when(en < BPW)
      def _():
        issue_gathers(en, b, _len_of(en))
    return carry

  lax.fori_loop(0, BPW // NBUF, chunk_body, 0)

  outstage[0, :] = acc_tot[...]
  outstage[1, :] = acc_cnt[...]
  pltpu.sync_copy(outstage, out_hbm.at[wid])


@jax.jit
def _hs_loss(inputs, target_path, target_path_len, target_code, W_embed, W_hs):
  mesh = plsc.VectorSubcoreMesh(core_axis_name="c", subcore_axis_name="s")
  parts = pl.kernel(
      _sc_body,
      out_type=jax.ShapeDtypeStruct((NW, 2, 16), jnp.float32),
      mesh=mesh,
      compiler_params=pltpu.CompilerParams(needs_layout_passes=False),
      scratch_types=[
          pltpu.VMEM((BPW,), jnp.int32),
          pltpu.VMEM((BPW * P,), jnp.int32),
          pltpu.VMEM((BPW * P,), jnp.int32),
          pltpu.VMEM((BPW,), jnp.int32),
          pltpu.VMEM((BPW, DIM), jnp.float32),
          pltpu.VMEM((NBUF, P, DIM), jnp.float32),
          pltpu.VMEM((16,), jnp.float32),
          pltpu.VMEM((16,), jnp.float32),
          pltpu.VMEM((2, 16), jnp.float32),
          pltpu.SemaphoreType.DMA,
          pltpu.SemaphoreType.DMA,
          [pltpu.SemaphoreType.DMA] * (2 * NBUF),
      ],
  )(inputs, target_path.reshape(-1), target_path_len, target_code.reshape(-1),
    W_embed, W_hs)
  total = jnp.sum(parts[:, 0, :])
  count = jnp.sum(parts[:, 1, :])
  return total / count


def kernel(inputs, target, target_path, target_path_len, target_code,
           W_embed, W_hs):
  del target
  return _hs_loss(inputs.astype(jnp.int32), target_path.astype(jnp.int32),
                  target_path_len.astype(jnp.int32),
                  target_code.astype(jnp.int32), W_embed, W_hs)
